# baseline (device time: 27250 ns/iter reference)
import jax
import jax.numpy as jnp
from jax import lax
from jax.experimental import pallas as pl
from jax.experimental.pallas import tpu as pltpu


def kernel(ids, E):
    T = ids.shape[0]
    V_local, D = E.shape

    def body(
        ids_smem,
        ids_vmem,
        e_ref,
        out_ref,
        gbuf,
        send_ref,
        recv_ref,
        gsem,
        send_sem,
        recv_sem,
    ):
        my_x = lax.axis_index("x")
        my_y = lax.axis_index("y")
        nbr = (1 - my_x, my_y)
        base = my_x * V_local

        def issue(i, carry):
            idx = ids_smem[i] - base
            idxc = jnp.clip(idx, 0, V_local - 1)
            pltpu.make_async_copy(
                e_ref.at[pl.ds(idxc, 1), :], gbuf.at[pl.ds(i, 1), :], gsem
            ).start()
            return carry

        lax.fori_loop(0, T, issue, 0)

        barrier = pltpu.get_barrier_semaphore()
        pl.semaphore_signal(
            barrier, inc=1, device_id=nbr, device_id_type=pl.DeviceIdType.MESH
        )
        pl.semaphore_wait(barrier, 1)

        def drain(i, carry):
            pltpu.make_async_copy(
                e_ref.at[pl.ds(0, 1), :], gbuf.at[pl.ds(i, 1), :], gsem
            ).wait()
            return carry

        lax.fori_loop(0, T, drain, 0)

        send_ref[...] = gbuf[...].astype(jnp.bfloat16)

        rdma = pltpu.make_async_remote_copy(
            src_ref=send_ref,
            dst_ref=recv_ref,
            send_sem=send_sem,
            recv_sem=recv_sem,
            device_id=nbr,
            device_id_type=pl.DeviceIdType.MESH,
        )
        rdma.start()
        rdma.wait()

        idx_v = ids_vmem[...] - base
        valid = (idx_v >= 0) & (idx_v < V_local)
        out_ref[...] = jnp.where(
            valid, gbuf[...], recv_ref[...].astype(jnp.float32)
        )

    return pl.pallas_call(
        body,
        out_shape=jax.ShapeDtypeStruct((T, D), jnp.float32),
        in_specs=[
            pl.BlockSpec(memory_space=pltpu.SMEM),
            pl.BlockSpec(memory_space=pltpu.VMEM),
            pl.BlockSpec(memory_space=pl.ANY),
        ],
        out_specs=pl.BlockSpec(memory_space=pltpu.VMEM),
        scratch_shapes=[
            pltpu.VMEM((T, D), jnp.float32),
            pltpu.VMEM((T, D), jnp.bfloat16),
            pltpu.VMEM((T, D), jnp.bfloat16),
            pltpu.SemaphoreType.DMA,
            pltpu.SemaphoreType.DMA,
            pltpu.SemaphoreType.DMA,
        ],
        compiler_params=pltpu.CompilerParams(collective_id=0),
    )(ids, ids.reshape(T, 1), E)


# device time: 19291 ns/iter; 1.4126x vs baseline; 1.4126x over previous
import jax
import jax.numpy as jnp
from jax import lax
from jax.experimental import pallas as pl
from jax.experimental.pallas import tpu as pltpu

C = 2
W = 128


def kernel(ids, E):
    T = ids.shape[0]
    V_local, D = E.shape
    assert C * W == D // 2

    def body(
        ids_ref,
        e_ref,
        out_ref,
        ebuf,
        sendx,
        recvx,
        sendy,
        recvy,
        esem,
        sx_send, sx_recv, sy_send, sy_recv,
    ):
        my_x = lax.axis_index("x")
        my_y = lax.axis_index("y")
        nbrx = (1 - my_x, my_y)
        nbry = (my_x, 1 - my_y)
        dbase = my_y * (D // 2)
        obase = (1 - my_y) * (D // 2)

        edma = []
        for c in range(C):
            dma = pltpu.make_async_copy(
                e_ref.at[:, pl.ds(dbase + c * W, W)],
                ebuf.at[c],
                esem.at[c],
            )
            dma.start()
            edma.append(dma)

        barrier = pltpu.get_barrier_semaphore()
        for nbr in (nbrx, nbry):
            pl.semaphore_signal(
                barrier,
                inc=1,
                device_id=nbr,
                device_id_type=pl.DeviceIdType.MESH,
            )
        pl.semaphore_wait(barrier, 2)

        idx = ids_ref[...] - my_x * V_local
        iota = lax.broadcasted_iota(jnp.int32, (T, V_local), 1)
        onehot = (idx == iota).astype(jnp.bfloat16)
        valid = (idx >= 0) & (idx < V_local)

        rdx = []
        for c in range(C):
            edma[c].wait()
            sendx[c] = jnp.dot(
                onehot,
                ebuf[c].astype(jnp.bfloat16),
                preferred_element_type=jnp.float32,
            ).astype(jnp.bfloat16)
            r = pltpu.make_async_remote_copy(
                src_ref=sendx.at[c],
                dst_ref=recvx.at[c],
                send_sem=sx_send.at[c],
                recv_sem=sx_recv.at[c],
                device_id=nbrx,
                device_id_type=pl.DeviceIdType.MESH,
            )
            r.start()
            rdx.append(r)

        rdy = []
        for c in range(C):
            rdx[c].wait_recv()
            comb = jnp.where(valid, sendx[c], recvx[c])
            sendy[c] = comb
            r = pltpu.make_async_remote_copy(
                src_ref=sendy.at[c],
                dst_ref=recvy.at[c],
                send_sem=sy_send.at[c],
                recv_sem=sy_recv.at[c],
                device_id=nbry,
                device_id_type=pl.DeviceIdType.MESH,
            )
            r.start()
            rdy.append(r)
            out_ref[:, pl.ds(dbase + c * W, W)] = comb.astype(jnp.float32)

        for c in range(C):
            rdy[c].wait_recv()
            out_ref[:, pl.ds(obase + c * W, W)] = recvy[c].astype(jnp.float32)

        for c in range(C):
            rdx[c].wait_send()
            rdy[c].wait_send()

    return pl.pallas_call(
        body,
        out_shape=jax.ShapeDtypeStruct((T, D), jnp.float32),
        in_specs=[
            pl.BlockSpec(memory_space=pltpu.VMEM),
            pl.BlockSpec(memory_space=pl.ANY),
        ],
        out_specs=pl.BlockSpec(memory_space=pltpu.VMEM),
        scratch_shapes=[
            pltpu.VMEM((C, V_local, W), jnp.float32),
            pltpu.VMEM((C, T, W), jnp.bfloat16),
            pltpu.VMEM((C, T, W), jnp.bfloat16),
            pltpu.VMEM((C, T, W), jnp.bfloat16),
            pltpu.VMEM((C, T, W), jnp.bfloat16),
            pltpu.SemaphoreType.DMA((C,)),
            pltpu.SemaphoreType.DMA((C,)),
            pltpu.SemaphoreType.DMA((C,)),
            pltpu.SemaphoreType.DMA((C,)),
            pltpu.SemaphoreType.DMA((C,)),
        ],
        compiler_params=pltpu.CompilerParams(collective_id=0),
    )(ids.reshape(T, 1), E)


# device time: 16951 ns/iter; 1.6076x vs baseline; 1.1380x over previous
import jax
import jax.numpy as jnp
from jax import lax
from jax.experimental import pallas as pl
from jax.experimental.pallas import tpu as pltpu

CC = 2


def kernel(ids, E):
    T = ids.shape[0]
    V_local, D = E.shape
    W = D // CC

    def body(ids_ref, e_ref, out_ref, sendx, recvx, sx_send, sx_recv):
        my_x = lax.axis_index("x")
        my_y = lax.axis_index("y")
        nbrx = (1 - my_x, my_y)

        barrier = pltpu.get_barrier_semaphore()
        pl.semaphore_signal(
            barrier, inc=1, device_id=nbrx, device_id_type=pl.DeviceIdType.MESH
        )
        pl.semaphore_wait(barrier, 1)

        idx = ids_ref[...] - my_x * V_local
        iota = lax.broadcasted_iota(jnp.int32, (T, V_local), 1)
        onehot = (idx == iota).astype(jnp.bfloat16)
        valid = (idx >= 0) & (idx < V_local)

        rdx = []
        for c in range(CC):
            sendx[c] = jnp.dot(
                onehot,
                e_ref[:, c * W : (c + 1) * W].astype(jnp.bfloat16),
                preferred_element_type=jnp.float32,
            ).astype(jnp.bfloat16)
            r = pltpu.make_async_remote_copy(
                src_ref=sendx.at[c],
                dst_ref=recvx.at[c],
                send_sem=sx_send.at[c],
                recv_sem=sx_recv.at[c],
                device_id=nbrx,
                device_id_type=pl.DeviceIdType.MESH,
            )
            r.start()
            rdx.append(r)

        for c in range(CC):
            rdx[c].wait_recv()
            comb = jnp.where(valid, sendx[c], recvx[c])
            out_ref[:, c * W : (c + 1) * W] = comb.astype(jnp.float32)

        for c in range(CC):
            rdx[c].wait_send()

    return pl.pallas_call(
        body,
        out_shape=jax.ShapeDtypeStruct((T, D), jnp.float32),
        in_specs=[
            pl.BlockSpec(memory_space=pltpu.VMEM),
            pl.BlockSpec(memory_space=pltpu.VMEM),
        ],
        out_specs=pl.BlockSpec(memory_space=pltpu.VMEM),
        scratch_shapes=[
            pltpu.VMEM((CC, T, D // CC), jnp.bfloat16),
            pltpu.VMEM((CC, T, D // CC), jnp.bfloat16),
            pltpu.SemaphoreType.DMA((CC,)),
            pltpu.SemaphoreType.DMA((CC,)),
        ],
        compiler_params=pltpu.CompilerParams(collective_id=0),
    )(ids.reshape(T, 1), E)
